# Initial kernel scaffold; baseline (speedup 1.0000x reference)
import jax
import jax.numpy as jnp
from jax.experimental import pallas as pl


def kernel(x, idx_labeled, labels, edge_index, W1, b1, W2, b2):
    raise NotImplementedError("write your pallas kernel here")


# speedup vs baseline: 16.2048x; 16.2048x over previous
import functools

import jax
import jax.numpy as jnp
from jax import lax
from jax.experimental import pallas as pl
from jax.experimental.pallas import tpu as pltpu
from jax.experimental.pallas import tpu_sc as plsc

NC = 2
NS = 16
LANES = 16
NW = NC * NS


def _make_deg_mask_kernel(N, E, NLABP):
    EPW = E // NW
    mesh = plsc.VectorSubcoreMesh(
        core_axis_name="c", subcore_axis_name="s",
        num_cores=NC, num_subcores=NS)

    @functools.partial(
        pl.kernel,
        out_type=(
            jax.ShapeDtypeStruct((NW, N), jnp.float32),
            jax.ShapeDtypeStruct((N,), jnp.float32),
        ),
        mesh=mesh,
        scratch_types=[
            pltpu.VMEM((EPW,), jnp.int32),
            pltpu.VMEM((N,), jnp.float32),
            pltpu.VMEM((NLABP,), jnp.int32),
            pltpu.VMEM((N,), jnp.float32),
        ],
        compiler_params=pltpu.CompilerParams(
            needs_layout_passes=False, use_tc_tiling_on_sc=False),
    )
    def deg_mask(dst_hbm, idxlab_hbm, hist_out, mask_out,
                 dstbuf, hist, labbuf, maskbuf):
        cid = lax.axis_index("c")
        sid = lax.axis_index("s")
        wid = cid * NS + sid
        ones = jnp.ones((LANES,), jnp.float32)
        zeros = jnp.zeros((LANES,), jnp.float32)

        pltpu.sync_copy(dst_hbm.at[pl.ds(wid * EPW, EPW)], dstbuf)

        @pl.loop(0, N // LANES)
        def _(i):
            hist[pl.ds(i * LANES, LANES)] = zeros

        @pl.loop(0, EPW // LANES)
        def _(i):
            idx = dstbuf[pl.ds(i * LANES, LANES)]
            plsc.addupdate_scatter(hist, [idx], ones)

        pltpu.sync_copy(hist, hist_out.at[wid])

        @pl.when(wid == 0)
        def _():
            pltpu.sync_copy(idxlab_hbm, labbuf)

            @pl.loop(0, N // LANES)
            def _(i):
                maskbuf[pl.ds(i * LANES, LANES)] = zeros

            @pl.loop(0, NLABP // LANES)
            def _(i):
                idx = labbuf[pl.ds(i * LANES, LANES)]
                plsc.store_scatter(maskbuf, [idx], ones)

            pltpu.sync_copy(maskbuf, mask_out)

    return deg_mask


def _make_spmm_kernel(N, E, W, K):
    EPW = E // NS
    CHUNKS = EPW // K
    assert CHUNKS * K == EPW and CHUNKS >= 4
    RPT = N // NS
    mesh = plsc.VectorSubcoreMesh(
        core_axis_name="c", subcore_axis_name="s",
        num_cores=NC, num_subcores=NS)

    @functools.partial(
        pl.kernel,
        out_type=jax.ShapeDtypeStruct((NC, N, W), jnp.float32),
        mesh=mesh,
        scratch_types=[
            pltpu.VMEM((EPW,), jnp.int32),
            pltpu.VMEM((CHUNKS, K), jnp.int32),
            pltpu.VMEM((K, W), jnp.float32),
            pltpu.VMEM((K, W), jnp.float32),
            pltpu.VMEM_SHARED((N, W), jnp.float32),
            pltpu.SemaphoreType.DMA,
            pltpu.SemaphoreType.DMA,
        ],
        compiler_params=pltpu.CompilerParams(
            needs_layout_passes=False, use_tc_tiling_on_sc=False),
    )
    def spmm(xs2_hbm, src_hbm, dst_hbm, zero_hbm, out_hbm,
             srcbuf, dstbuf, rows0, rows1, acc, sem0, sem1):
        cid = lax.axis_index("c")
        sid = lax.axis_index("s")
        r0 = sid * RPT

        pltpu.sync_copy(zero_hbm.at[pl.ds(r0, RPT)], acc.at[pl.ds(r0, RPT)])

        pltpu.sync_copy(src_hbm.at[pl.ds(sid * EPW, EPW)], srcbuf)
        pltpu.sync_copy(dst_hbm.at[pl.ds(sid * CHUNKS, CHUNKS)], dstbuf)

        @pl.loop(0, EPW // LANES)
        def _(i):
            v = srcbuf[pl.ds(i * LANES, LANES)]
            srcbuf[pl.ds(i * LANES, LANES)] = v * 2 + cid

        plsc.subcore_barrier()

        def fire(c, rows, sem):
            pltpu.async_copy(xs2_hbm.at[srcbuf.at[pl.ds(c * K, K)]], rows, sem)

        def wait(c, rows, sem):
            pltpu.make_async_copy(
                xs2_hbm.at[srcbuf.at[pl.ds(c * K, K)]], rows, sem).wait()

        def scatter(c, rows):
            pltpu.sync_copy(rows, acc.at[dstbuf.at[c]], add=True)

        fire(0, rows0, sem0)

        if CHUNKS % 2 == 1:
            loop_t, tail2 = (CHUNKS - 1) // 2, False
        else:
            loop_t, tail2 = CHUNKS // 2 - 1, True

        @pl.loop(0, loop_t)
        def _(t):
            a = 2 * t
            fire(a + 1, rows1, sem1)
            wait(a, rows0, sem0)
            scatter(a, rows0)
            fire(a + 2, rows0, sem0)
            wait(a + 1, rows1, sem1)
            scatter(a + 1, rows1)

        if tail2:
            fire(CHUNKS - 1, rows1, sem1)
            wait(CHUNKS - 2, rows0, sem0)
            scatter(CHUNKS - 2, rows0)
            wait(CHUNKS - 1, rows1, sem1)
            scatter(CHUNKS - 1, rows1)
        else:
            wait(CHUNKS - 1, rows0, sem0)
            scatter(CHUNKS - 1, rows0)

        plsc.subcore_barrier()
        pltpu.sync_copy(acc.at[pl.ds(r0, RPT)], out_hbm.at[cid, pl.ds(r0, RPT)])

    return spmm


def _scale_build_xs(histT, mask2, lab2, x, C, FP1, BN):
    N, F0 = x.shape
    PAD1 = FP1 - F0 - C

    def body(hist_ref, mask_ref, lab_ref, x_ref, xs_ref, rd_ref):
        deg = jnp.sum(hist_ref[...], axis=1, keepdims=True) + 1.0
        rd = lax.rsqrt(deg)
        iota = lax.broadcasted_iota(jnp.int32, (BN, C), 1)
        oh = (lab_ref[...] == iota).astype(jnp.float32) * mask_ref[...] * rd
        xs_ref[...] = jnp.concatenate(
            [x_ref[...] * rd, oh, jnp.zeros((BN, PAD1), jnp.float32)], axis=1)
        rd_ref[...] = rd

    return pl.pallas_call(
        body,
        grid=(N // BN,),
        in_specs=[
            pl.BlockSpec((BN, histT.shape[1]), lambda i: (i, 0)),
            pl.BlockSpec((BN, 1), lambda i: (i, 0)),
            pl.BlockSpec((BN, 1), lambda i: (i, 0)),
            pl.BlockSpec((BN, F0), lambda i: (i, 0)),
        ],
        out_specs=[
            pl.BlockSpec((BN, FP1), lambda i: (i, 0)),
            pl.BlockSpec((BN, 1), lambda i: (i, 0)),
        ],
        out_shape=[
            jax.ShapeDtypeStruct((N, FP1), jnp.float32),
            jax.ShapeDtypeStruct((N, 1), jnp.float32),
        ],
    )(histT, mask2, lab2, x)


def _mlp_mid(lo, hi, rd, W1p, b1r, W2, FP2, BN):
    N, W = lo.shape
    H = W1p.shape[1]
    C = W2.shape[1]
    PAD2 = FP2 - C

    def body(lo_ref, hi_ref, rd_ref, w1_ref, b1_ref, w2_ref, out_ref):
        rdv = rd_ref[...]
        p = jnp.concatenate([lo_ref[...], hi_ref[...]], axis=1) * rdv
        h = jnp.maximum(
            jnp.dot(p, w1_ref[...], preferred_element_type=jnp.float32)
            + b1_ref[...], 0.0)
        g = jnp.dot(h, w2_ref[...], preferred_element_type=jnp.float32)
        g = g * rdv
        out_ref[...] = jnp.concatenate(
            [g, jnp.zeros((BN, PAD2), jnp.float32)], axis=1)

    return pl.pallas_call(
        body,
        grid=(N // BN,),
        in_specs=[
            pl.BlockSpec((BN, W), lambda i: (i, 0)),
            pl.BlockSpec((BN, W), lambda i: (i, 0)),
            pl.BlockSpec((BN, 1), lambda i: (i, 0)),
            pl.BlockSpec((2 * W, H), lambda i: (0, 0)),
            pl.BlockSpec((1, H), lambda i: (0, 0)),
            pl.BlockSpec((H, C), lambda i: (0, 0)),
        ],
        out_specs=pl.BlockSpec((BN, FP2), lambda i: (i, 0)),
        out_shape=jax.ShapeDtypeStruct((N, FP2), jnp.float32),
    )(lo, hi, rd, W1p, b1r, W2)


def _final_logsoftmax(lo, hi, rd, b2r, C, BN):
    N, W = lo.shape

    def body(lo_ref, hi_ref, rd_ref, b2_ref, out_ref):
        s = jnp.concatenate([lo_ref[...], hi_ref[...]], axis=1) * rd_ref[...]
        o = s[:, :C] + b2_ref[...]
        m = jnp.max(o, axis=1, keepdims=True)
        e = jnp.exp(o - m)
        lse = jnp.log(jnp.sum(e, axis=1, keepdims=True))
        out_ref[...] = o - m - lse

    return pl.pallas_call(
        body,
        grid=(N // BN,),
        in_specs=[
            pl.BlockSpec((BN, W), lambda i: (i, 0)),
            pl.BlockSpec((BN, W), lambda i: (i, 0)),
            pl.BlockSpec((BN, 1), lambda i: (i, 0)),
            pl.BlockSpec((1, C), lambda i: (0, 0)),
        ],
        out_specs=pl.BlockSpec((BN, C), lambda i: (i, 0)),
        out_shape=jax.ShapeDtypeStruct((N, C), jnp.float32),
    )(lo, hi, rd, b2r)


def kernel(x, idx_labeled, labels, edge_index, W1, b1, W2, b2):
    N, F0 = x.shape
    C = W2.shape[1]
    H = W1.shape[1]
    E = edge_index.shape[1]
    NLAB = idx_labeled.shape[0]

    F1 = F0 + C
    FP1 = 192
    FP2 = 64
    K = 80
    BN = 1000
    NLABP = ((NLAB + LANES - 1) // LANES) * LANES

    src = edge_index[0]
    dst = edge_index[1]
    dst2d = dst.reshape(E // K, K)
    idxlab_p = jnp.concatenate(
        [idx_labeled, jnp.broadcast_to(idx_labeled[:1], (NLABP - NLAB,))])

    deg_mask = _make_deg_mask_kernel(N, E, NLABP)
    hist, mask = deg_mask(dst, idxlab_p)

    xs, rd = _scale_build_xs(
        hist.T, mask.reshape(N, 1), labels.reshape(N, 1), x, C, FP1, BN)

    spmm1 = _make_spmm_kernel(N, E, FP1 // 2, K)
    acc1 = spmm1(xs.reshape(2 * N, FP1 // 2), src, dst2d,
                 jnp.zeros((N, FP1 // 2), jnp.float32))

    W1p = jnp.pad(W1, ((0, FP1 - F1), (0, 0)))
    gs = _mlp_mid(acc1[0], acc1[1], rd, W1p, b1.reshape(1, H), W2, FP2, BN)

    spmm2 = _make_spmm_kernel(N, E, FP2 // 2, K)
    acc2 = spmm2(gs.reshape(2 * N, FP2 // 2), src, dst2d,
                 jnp.zeros((N, FP2 // 2), jnp.float32))

    return _final_logsoftmax(acc2[0], acc2[1], rd, b2.reshape(1, C), C, BN)


# speedup vs baseline: 18.1953x; 1.1228x over previous
import functools

import jax
import jax.numpy as jnp
from jax import lax
from jax.experimental import pallas as pl
from jax.experimental.pallas import tpu as pltpu
from jax.experimental.pallas import tpu_sc as plsc

NC = 2
NS = 16
LANES = 16
NW = NC * NS


def _make_deg_mask_kernel(N, E, NLABP):
    EPW = E // NW
    mesh = plsc.VectorSubcoreMesh(
        core_axis_name="c", subcore_axis_name="s",
        num_cores=NC, num_subcores=NS)

    @functools.partial(
        pl.kernel,
        out_type=(
            jax.ShapeDtypeStruct((NW, N), jnp.float32),
            jax.ShapeDtypeStruct((N,), jnp.float32),
        ),
        mesh=mesh,
        scratch_types=[
            pltpu.VMEM((EPW,), jnp.int32),
            pltpu.VMEM((N,), jnp.float32),
            pltpu.VMEM((NLABP,), jnp.int32),
            pltpu.VMEM((N,), jnp.float32),
        ],
        compiler_params=pltpu.CompilerParams(
            needs_layout_passes=False, use_tc_tiling_on_sc=False),
    )
    def deg_mask(dst_hbm, idxlab_hbm, hist_out, mask_out,
                 dstbuf, hist, labbuf, maskbuf):
        cid = lax.axis_index("c")
        sid = lax.axis_index("s")
        wid = cid * NS + sid
        ones = jnp.ones((LANES,), jnp.float32)
        zeros = jnp.zeros((LANES,), jnp.float32)

        pltpu.sync_copy(dst_hbm.at[pl.ds(wid * EPW, EPW)], dstbuf)

        @pl.loop(0, N // LANES)
        def _(i):
            hist[pl.ds(i * LANES, LANES)] = zeros

        @pl.loop(0, EPW // LANES)
        def _(i):
            idx = dstbuf[pl.ds(i * LANES, LANES)]
            plsc.addupdate_scatter(hist, [idx], ones)

        pltpu.sync_copy(hist, hist_out.at[wid])

        @pl.when(wid == 0)
        def _():
            pltpu.sync_copy(idxlab_hbm, labbuf)

            @pl.loop(0, N // LANES)
            def _(i):
                maskbuf[pl.ds(i * LANES, LANES)] = zeros

            @pl.loop(0, NLABP // LANES)
            def _(i):
                idx = labbuf[pl.ds(i * LANES, LANES)]
                plsc.store_scatter(maskbuf, [idx], ones)

            pltpu.sync_copy(maskbuf, mask_out)

    return deg_mask


def _make_spmm_kernel(N, E, W, K):
    EPW = E // NS
    CHUNKS = EPW // K
    assert CHUNKS * K == EPW and CHUNKS >= 7 and (CHUNKS - 4) % 3 == 0
    RPT = N // NS
    mesh = plsc.VectorSubcoreMesh(
        core_axis_name="c", subcore_axis_name="s",
        num_cores=NC, num_subcores=NS)

    @functools.partial(
        pl.kernel,
        out_type=jax.ShapeDtypeStruct((NC, N, W), jnp.float32),
        mesh=mesh,
        scratch_types=[
            pltpu.VMEM((EPW,), jnp.int32),
            pltpu.VMEM((CHUNKS, K), jnp.int32),
            pltpu.VMEM((K, W), jnp.float32),
            pltpu.VMEM((K, W), jnp.float32),
            pltpu.VMEM((K, W), jnp.float32),
            pltpu.SemaphoreType.DMA,
            pltpu.SemaphoreType.DMA,
            pltpu.SemaphoreType.DMA,
            pltpu.SemaphoreType.DMA,
            pltpu.SemaphoreType.DMA,
            pltpu.SemaphoreType.DMA,
            pltpu.VMEM_SHARED((N, W), jnp.float32),
        ],
        compiler_params=pltpu.CompilerParams(
            needs_layout_passes=False, use_tc_tiling_on_sc=False),
    )
    def spmm(xs2_hbm, src_hbm, dst_hbm, zero_hbm, out_hbm,
             srcbuf, dstbuf, rows0, rows1, rows2,
             gsem0, gsem1, gsem2, ssem0, ssem1, ssem2, acc):
        cid = lax.axis_index("c")
        sid = lax.axis_index("s")
        r0 = sid * RPT
        rows = (rows0, rows1, rows2)
        gsem = (gsem0, gsem1, gsem2)
        ssem = (ssem0, ssem1, ssem2)

        pltpu.sync_copy(zero_hbm.at[pl.ds(r0, RPT)], acc.at[pl.ds(r0, RPT)])

        pltpu.sync_copy(src_hbm.at[pl.ds(sid * EPW, EPW)], srcbuf)
        pltpu.sync_copy(dst_hbm.at[pl.ds(sid * CHUNKS, CHUNKS)], dstbuf)

        @pl.loop(0, EPW // LANES)
        def _(i):
            v = srcbuf[pl.ds(i * LANES, LANES)]
            srcbuf[pl.ds(i * LANES, LANES)] = v * 2 + cid

        plsc.subcore_barrier()

        def fire_g(c, j):
            pltpu.async_copy(
                xs2_hbm.at[srcbuf.at[pl.ds(c * K, K)]], rows[j], gsem[j])

        def wait_g(c, j):
            pltpu.make_async_copy(
                xs2_hbm.at[srcbuf.at[pl.ds(c * K, K)]], rows[j], gsem[j]).wait()

        def fire_s(c, j):
            pltpu.async_copy(rows[j], acc.at[dstbuf.at[c]], ssem[j], add=True)

        def wait_s(c, j):
            pltpu.make_async_copy(
                rows[j], acc.at[dstbuf.at[c]], ssem[j]).wait()

        fire_g(0, 0)
        fire_g(1, 1)
        wait_g(0, 0)
        fire_s(0, 0)
        fire_g(2, 2)
        wait_g(1, 1)
        fire_s(1, 1)
        wait_s(0, 0)
        fire_g(3, 0)

        @pl.loop(0, (CHUNKS - 4) // 3)
        def _(t):
            base = 3 * t + 2
            for k in range(3):
                c = base + k
                j = (2 + k) % 3
                jp = (j + 2) % 3
                wait_g(c, j)
                fire_s(c, j)
                wait_s(c - 1, jp)
                fire_g(c + 2, jp)

        for c in (CHUNKS - 2, CHUNKS - 1):
            j = c % 3
            wait_g(c, j)
            fire_s(c, j)
            wait_s(c - 1, (c - 1) % 3)
        wait_s(CHUNKS - 1, (CHUNKS - 1) % 3)

        plsc.subcore_barrier()
        pltpu.sync_copy(acc.at[pl.ds(r0, RPT)], out_hbm.at[cid, pl.ds(r0, RPT)])

    return spmm


def _scale_build_xs(histT, mask2, lab2, x, C, FP1, BN):
    N, F0 = x.shape
    PAD1 = FP1 - F0 - C

    def body(hist_ref, mask_ref, lab_ref, x_ref, xs_ref, rd_ref):
        deg = jnp.sum(hist_ref[...], axis=1, keepdims=True) + 1.0
        rd = lax.rsqrt(deg)
        iota = lax.broadcasted_iota(jnp.int32, (BN, C), 1)
        oh = (lab_ref[...] == iota).astype(jnp.float32) * mask_ref[...] * rd
        xs_ref[...] = jnp.concatenate(
            [x_ref[...] * rd, oh, jnp.zeros((BN, PAD1), jnp.float32)], axis=1)
        rd_ref[...] = rd

    return pl.pallas_call(
        body,
        grid=(N // BN,),
        in_specs=[
            pl.BlockSpec((BN, histT.shape[1]), lambda i: (i, 0)),
            pl.BlockSpec((BN, 1), lambda i: (i, 0)),
            pl.BlockSpec((BN, 1), lambda i: (i, 0)),
            pl.BlockSpec((BN, F0), lambda i: (i, 0)),
        ],
        out_specs=[
            pl.BlockSpec((BN, FP1), lambda i: (i, 0)),
            pl.BlockSpec((BN, 1), lambda i: (i, 0)),
        ],
        out_shape=[
            jax.ShapeDtypeStruct((N, FP1), jnp.float32),
            jax.ShapeDtypeStruct((N, 1), jnp.float32),
        ],
    )(histT, mask2, lab2, x)


def _mlp_mid(lo, hi, rd, W1p, b1r, W2, FP2, BN):
    N, W = lo.shape
    H = W1p.shape[1]
    C = W2.shape[1]
    PAD2 = FP2 - C

    def body(lo_ref, hi_ref, rd_ref, w1_ref, b1_ref, w2_ref, out_ref):
        rdv = rd_ref[...]
        p = jnp.concatenate([lo_ref[...], hi_ref[...]], axis=1) * rdv
        h = jnp.maximum(
            jnp.dot(p, w1_ref[...], preferred_element_type=jnp.float32)
            + b1_ref[...], 0.0)
        g = jnp.dot(h, w2_ref[...], preferred_element_type=jnp.float32)
        g = g * rdv
        out_ref[...] = jnp.concatenate(
            [g, jnp.zeros((BN, PAD2), jnp.float32)], axis=1)

    return pl.pallas_call(
        body,
        grid=(N // BN,),
        in_specs=[
            pl.BlockSpec((BN, W), lambda i: (i, 0)),
            pl.BlockSpec((BN, W), lambda i: (i, 0)),
            pl.BlockSpec((BN, 1), lambda i: (i, 0)),
            pl.BlockSpec((2 * W, H), lambda i: (0, 0)),
            pl.BlockSpec((1, H), lambda i: (0, 0)),
            pl.BlockSpec((H, C), lambda i: (0, 0)),
        ],
        out_specs=pl.BlockSpec((BN, FP2), lambda i: (i, 0)),
        out_shape=jax.ShapeDtypeStruct((N, FP2), jnp.float32),
    )(lo, hi, rd, W1p, b1r, W2)


def _final_logsoftmax(lo, hi, rd, b2r, C, BN):
    N, W = lo.shape

    def body(lo_ref, hi_ref, rd_ref, b2_ref, out_ref):
        s = jnp.concatenate([lo_ref[...], hi_ref[...]], axis=1) * rd_ref[...]
        o = s[:, :C] + b2_ref[...]
        m = jnp.max(o, axis=1, keepdims=True)
        e = jnp.exp(o - m)
        lse = jnp.log(jnp.sum(e, axis=1, keepdims=True))
        out_ref[...] = o - m - lse

    return pl.pallas_call(
        body,
        grid=(N // BN,),
        in_specs=[
            pl.BlockSpec((BN, W), lambda i: (i, 0)),
            pl.BlockSpec((BN, W), lambda i: (i, 0)),
            pl.BlockSpec((BN, 1), lambda i: (i, 0)),
            pl.BlockSpec((1, C), lambda i: (0, 0)),
        ],
        out_specs=pl.BlockSpec((BN, C), lambda i: (i, 0)),
        out_shape=jax.ShapeDtypeStruct((N, C), jnp.float32),
    )(lo, hi, rd, b2r)


def kernel(x, idx_labeled, labels, edge_index, W1, b1, W2, b2):
    N, F0 = x.shape
    C = W2.shape[1]
    H = W1.shape[1]
    E = edge_index.shape[1]
    NLAB = idx_labeled.shape[0]

    F1 = F0 + C
    FP1 = 192
    FP2 = 64
    K = 80
    BN = 1000
    NLABP = ((NLAB + LANES - 1) // LANES) * LANES

    src = edge_index[0]
    dst = edge_index[1]
    dst2d = dst.reshape(E // K, K)
    idxlab_p = jnp.concatenate(
        [idx_labeled, jnp.broadcast_to(idx_labeled[:1], (NLABP - NLAB,))])

    deg_mask = _make_deg_mask_kernel(N, E, NLABP)
    hist, mask = deg_mask(dst, idxlab_p)

    xs, rd = _scale_build_xs(
        hist.T, mask.reshape(N, 1), labels.reshape(N, 1), x, C, FP1, BN)

    spmm1 = _make_spmm_kernel(N, E, FP1 // 2, K)
    acc1 = spmm1(xs.reshape(2 * N, FP1 // 2), src, dst2d,
                 jnp.zeros((N, FP1 // 2), jnp.float32))

    W1p = jnp.pad(W1, ((0, FP1 - F1), (0, 0)))
    gs = _mlp_mid(acc1[0], acc1[1], rd, W1p, b1.reshape(1, H), W2, FP2, BN)

    spmm2 = _make_spmm_kernel(N, E, FP2 // 2, K)
    acc2 = spmm2(gs.reshape(2 * N, FP2 // 2), src, dst2d,
                 jnp.zeros((N, FP2 // 2), jnp.float32))

    return _final_logsoftmax(acc2[0], acc2[1], rd, b2.reshape(1, C), C, BN)


# speedup vs baseline: 18.5592x; 1.0200x over previous
import functools

import jax
import jax.numpy as jnp
from jax import lax
from jax.experimental import pallas as pl
from jax.experimental.pallas import tpu as pltpu
from jax.experimental.pallas import tpu_sc as plsc

NC = 2
NS = 16
LANES = 16
NW = NC * NS


def _make_deg_mask_kernel(N, E, NLABP):
    EPW = E // NW
    HC = 128
    HR = (N + HC - 1) // HC
    HR = ((HR + NS - 1) // NS) * NS
    RPT = HR // NS
    mesh = plsc.VectorSubcoreMesh(
        core_axis_name="c", subcore_axis_name="s",
        num_cores=NC, num_subcores=NS)

    @functools.partial(
        pl.kernel,
        out_type=(
            jax.ShapeDtypeStruct((NC, HR, HC), jnp.float32),
            jax.ShapeDtypeStruct((N,), jnp.float32),
        ),
        mesh=mesh,
        scratch_types=[
            pltpu.VMEM((EPW,), jnp.int32),
            pltpu.VMEM((HR, HC), jnp.float32),
            pltpu.VMEM((HR,), jnp.int32),
            pltpu.VMEM((NLABP,), jnp.int32),
            pltpu.VMEM((N,), jnp.float32),
            pltpu.VMEM_SHARED((HR, HC), jnp.float32),
        ],
        compiler_params=pltpu.CompilerParams(
            needs_layout_passes=False, use_tc_tiling_on_sc=False),
    )
    def deg_mask(dst_hbm, idxlab_hbm, deg_out, mask_out,
                 dstbuf, hist, rowids, labbuf, maskbuf, dacc):
        cid = lax.axis_index("c")
        sid = lax.axis_index("s")
        wid = cid * NS + sid
        ones = jnp.ones((LANES,), jnp.float32)
        zeros = jnp.zeros((LANES,), jnp.float32)
        iota = lax.iota(jnp.int32, LANES)

        pltpu.sync_copy(dst_hbm.at[pl.ds(wid * EPW, EPW)], dstbuf)

        @pl.loop(0, HR)
        def _(r):
            @pl.loop(0, HC // LANES)
            def _(i):
                hist[r, pl.ds(i * LANES, LANES)] = zeros

        @pl.loop(0, HR // LANES)
        def _(i):
            rowids[pl.ds(i * LANES, LANES)] = iota + i * LANES

        pltpu.sync_copy(hist.at[pl.ds(sid * RPT, RPT)],
                        dacc.at[pl.ds(sid * RPT, RPT)])

        @pl.loop(0, EPW // LANES)
        def _(i):
            idx = dstbuf[pl.ds(i * LANES, LANES)]
            plsc.addupdate_scatter(
                hist, [lax.shift_right_logical(idx, 7),
                       lax.bitwise_and(idx, HC - 1)], ones)

        plsc.subcore_barrier()
        pltpu.sync_copy(hist, dacc.at[rowids], add=True)
        plsc.subcore_barrier()
        pltpu.sync_copy(dacc.at[pl.ds(sid * RPT, RPT)],
                        deg_out.at[cid, pl.ds(sid * RPT, RPT)])

        @pl.when(wid == 0)
        def _():
            pltpu.sync_copy(idxlab_hbm, labbuf)

            @pl.loop(0, N // LANES)
            def _(i):
                maskbuf[pl.ds(i * LANES, LANES)] = zeros

            @pl.loop(0, NLABP // LANES)
            def _(i):
                idx = labbuf[pl.ds(i * LANES, LANES)]
                plsc.store_scatter(maskbuf, [idx], ones)

            pltpu.sync_copy(maskbuf, mask_out)

    return deg_mask


def _make_spmm_kernel(N, E, W, K):
    EPW = E // NS
    CHUNKS = EPW // K
    assert CHUNKS * K == EPW and CHUNKS >= 7 and (CHUNKS - 4) % 3 == 0
    RPT = N // NS
    mesh = plsc.VectorSubcoreMesh(
        core_axis_name="c", subcore_axis_name="s",
        num_cores=NC, num_subcores=NS)

    @functools.partial(
        pl.kernel,
        out_type=jax.ShapeDtypeStruct((NC, N, W), jnp.float32),
        mesh=mesh,
        scratch_types=[
            pltpu.VMEM((EPW,), jnp.int32),
            pltpu.VMEM((CHUNKS, K), jnp.int32),
            pltpu.VMEM((K, W), jnp.float32),
            pltpu.VMEM((K, W), jnp.float32),
            pltpu.VMEM((K, W), jnp.float32),
            pltpu.SemaphoreType.DMA,
            pltpu.SemaphoreType.DMA,
            pltpu.SemaphoreType.DMA,
            pltpu.SemaphoreType.DMA,
            pltpu.SemaphoreType.DMA,
            pltpu.SemaphoreType.DMA,
            pltpu.VMEM_SHARED((N, W), jnp.float32),
        ],
        compiler_params=pltpu.CompilerParams(
            needs_layout_passes=False, use_tc_tiling_on_sc=False),
    )
    def spmm(xs2_hbm, src_hbm, dst_hbm, out_hbm,
             srcbuf, dstbuf, rows0, rows1, rows2,
             gsem0, gsem1, gsem2, ssem0, ssem1, ssem2, acc):
        cid = lax.axis_index("c")
        sid = lax.axis_index("s")
        r0 = sid * RPT
        rows = (rows0, rows1, rows2)
        gsem = (gsem0, gsem1, gsem2)
        ssem = (ssem0, ssem1, ssem2)

        zeros = jnp.zeros((LANES,), jnp.float32)

        @pl.loop(0, K)
        def _(r):
            @pl.loop(0, W // LANES)
            def _(i):
                rows0[r, pl.ds(i * LANES, LANES)] = zeros

        for kblk in range(RPT // K):
            pltpu.sync_copy(rows0, acc.at[pl.ds(r0 + kblk * K, K)])
        _rem = RPT - (RPT // K) * K
        if _rem:
            pltpu.sync_copy(rows0.at[pl.ds(0, _rem)],
                            acc.at[pl.ds(r0 + (RPT // K) * K, _rem)])

        pltpu.sync_copy(src_hbm.at[pl.ds(sid * EPW, EPW)], srcbuf)
        pltpu.sync_copy(dst_hbm.at[pl.ds(sid * CHUNKS, CHUNKS)], dstbuf)

        @pl.loop(0, EPW // LANES)
        def _(i):
            v = srcbuf[pl.ds(i * LANES, LANES)]
            srcbuf[pl.ds(i * LANES, LANES)] = v * 2 + cid

        plsc.subcore_barrier()

        def fire_g(c, j):
            pltpu.async_copy(
                xs2_hbm.at[srcbuf.at[pl.ds(c * K, K)]], rows[j], gsem[j])

        def wait_g(c, j):
            pltpu.make_async_copy(
                xs2_hbm.at[srcbuf.at[pl.ds(c * K, K)]], rows[j], gsem[j]).wait()

        def fire_s(c, j):
            pltpu.async_copy(rows[j], acc.at[dstbuf.at[c]], ssem[j], add=True)

        def wait_s(c, j):
            pltpu.make_async_copy(
                rows[j], acc.at[dstbuf.at[c]], ssem[j]).wait()

        fire_g(0, 0)
        fire_g(1, 1)
        wait_g(0, 0)
        fire_s(0, 0)
        fire_g(2, 2)
        wait_g(1, 1)
        fire_s(1, 1)
        wait_s(0, 0)
        fire_g(3, 0)

        @pl.loop(0, (CHUNKS - 4) // 3)
        def _(t):
            base = 3 * t + 2
            for k in range(3):
                c = base + k
                j = (2 + k) % 3
                jp = (j + 2) % 3
                wait_g(c, j)
                fire_s(c, j)
                wait_s(c - 1, jp)
                fire_g(c + 2, jp)

        for c in (CHUNKS - 2, CHUNKS - 1):
            j = c % 3
            wait_g(c, j)
            fire_s(c, j)
            wait_s(c - 1, (c - 1) % 3)
        wait_s(CHUNKS - 1, (CHUNKS - 1) % 3)

        plsc.subcore_barrier()
        pltpu.sync_copy(acc.at[pl.ds(r0, RPT)], out_hbm.at[cid, pl.ds(r0, RPT)])

    return spmm


def _scale_build_xs(deg2, mask2, lab2, x, C, FP1, BN):
    N, F0 = x.shape
    PAD1 = FP1 - F0 - C

    def body(deg_ref, mask_ref, lab_ref, x_ref, xs_ref, rd_ref):
        rd = lax.rsqrt(deg_ref[...] + 1.0)
        iota = lax.broadcasted_iota(jnp.int32, (BN, C), 1)
        oh = (lab_ref[...] == iota).astype(jnp.float32) * mask_ref[...] * rd
        xs_ref[...] = jnp.concatenate(
            [x_ref[...] * rd, oh, jnp.zeros((BN, PAD1), jnp.float32)], axis=1)
        rd_ref[...] = rd

    return pl.pallas_call(
        body,
        grid=(N // BN,),
        in_specs=[
            pl.BlockSpec((BN, 1), lambda i: (i, 0)),
            pl.BlockSpec((BN, 1), lambda i: (i, 0)),
            pl.BlockSpec((BN, 1), lambda i: (i, 0)),
            pl.BlockSpec((BN, F0), lambda i: (i, 0)),
        ],
        out_specs=[
            pl.BlockSpec((BN, FP1), lambda i: (i, 0)),
            pl.BlockSpec((BN, 1), lambda i: (i, 0)),
        ],
        out_shape=[
            jax.ShapeDtypeStruct((N, FP1), jnp.float32),
            jax.ShapeDtypeStruct((N, 1), jnp.float32),
        ],
    )(deg2, mask2, lab2, x)


def _mlp_mid(lo, hi, rd, W1p, b1r, W2, FP2, BN):
    N, W = lo.shape
    H = W1p.shape[1]
    C = W2.shape[1]
    PAD2 = FP2 - C

    def body(lo_ref, hi_ref, rd_ref, w1_ref, b1_ref, w2_ref, out_ref):
        rdv = rd_ref[...]
        p = jnp.concatenate([lo_ref[...], hi_ref[...]], axis=1) * rdv
        h = jnp.maximum(
            jnp.dot(p, w1_ref[...], preferred_element_type=jnp.float32)
            + b1_ref[...], 0.0)
        g = jnp.dot(h, w2_ref[...], preferred_element_type=jnp.float32)
        g = g * rdv
        out_ref[...] = jnp.concatenate(
            [g, jnp.zeros((BN, PAD2), jnp.float32)], axis=1)

    return pl.pallas_call(
        body,
        grid=(N // BN,),
        in_specs=[
            pl.BlockSpec((BN, W), lambda i: (i, 0)),
            pl.BlockSpec((BN, W), lambda i: (i, 0)),
            pl.BlockSpec((BN, 1), lambda i: (i, 0)),
            pl.BlockSpec((2 * W, H), lambda i: (0, 0)),
            pl.BlockSpec((1, H), lambda i: (0, 0)),
            pl.BlockSpec((H, C), lambda i: (0, 0)),
        ],
        out_specs=pl.BlockSpec((BN, FP2), lambda i: (i, 0)),
        out_shape=jax.ShapeDtypeStruct((N, FP2), jnp.float32),
    )(lo, hi, rd, W1p, b1r, W2)


def _final_logsoftmax(lo, hi, rd, b2r, C, BN):
    N, W = lo.shape

    def body(lo_ref, hi_ref, rd_ref, b2_ref, out_ref):
        s = jnp.concatenate([lo_ref[...], hi_ref[...]], axis=1) * rd_ref[...]
        o = s[:, :C] + b2_ref[...]
        m = jnp.max(o, axis=1, keepdims=True)
        e = jnp.exp(o - m)
        lse = jnp.log(jnp.sum(e, axis=1, keepdims=True))
        out_ref[...] = o - m - lse

    return pl.pallas_call(
        body,
        grid=(N // BN,),
        in_specs=[
            pl.BlockSpec((BN, W), lambda i: (i, 0)),
            pl.BlockSpec((BN, W), lambda i: (i, 0)),
            pl.BlockSpec((BN, 1), lambda i: (i, 0)),
            pl.BlockSpec((1, C), lambda i: (0, 0)),
        ],
        out_specs=pl.BlockSpec((BN, C), lambda i: (i, 0)),
        out_shape=jax.ShapeDtypeStruct((N, C), jnp.float32),
    )(lo, hi, rd, b2r)


def kernel(x, idx_labeled, labels, edge_index, W1, b1, W2, b2):
    N, F0 = x.shape
    C = W2.shape[1]
    H = W1.shape[1]
    E = edge_index.shape[1]
    NLAB = idx_labeled.shape[0]

    F1 = F0 + C
    FP1 = 192
    FP2 = 64
    K = 80
    BN = 1000
    NLABP = ((NLAB + LANES - 1) // LANES) * LANES

    src = edge_index[0]
    dst = edge_index[1]
    dst2d = dst.reshape(E // K, K)
    idxlab_p = jnp.concatenate(
        [idx_labeled, jnp.broadcast_to(idx_labeled[:1], (NLABP - NLAB,))])

    deg_mask = _make_deg_mask_kernel(N, E, NLABP)
    degp, mask = deg_mask(dst, idxlab_p)
    deg2 = (degp[0] + degp[1]).reshape(-1)[:N].reshape(N, 1)

    xs, rd = _scale_build_xs(
        deg2, mask.reshape(N, 1), labels.reshape(N, 1), x, C, FP1, BN)

    spmm1 = _make_spmm_kernel(N, E, FP1 // 2, K)
    acc1 = spmm1(xs.reshape(2 * N, FP1 // 2), src, dst2d)

    W1p = jnp.pad(W1, ((0, FP1 - F1), (0, 0)))
    gs = _mlp_mid(acc1[0], acc1[1], rd, W1p, b1.reshape(1, H), W2, FP2, BN)

    spmm2 = _make_spmm_kernel(N, E, FP2 // 2, K)
    acc2 = spmm2(gs.reshape(2 * N, FP2 // 2), src, dst2d)

    return _final_logsoftmax(acc2[0], acc2[1], rd, b2.reshape(1, C), C, BN)
